# baseline (speedup 1.0000x reference)
import functools

import jax
import jax.numpy as jnp
from jax import lax
from jax.experimental import pallas as pl
from jax.experimental.pallas import tpu as pltpu
from jax.experimental.pallas import tpu_sc as plsc

_EMBED = 32
_HALF = 256


def _gather_kernel(uid_hbm, mid_hbm, utab_hbm, mtab_hbm, gu_hbm, gm_hbm,
                   uid_v, mid_v, du_v, dm_v, sem,
                   *, b_per_w, num_cores):
    wid = lax.axis_index("s") * num_cores + lax.axis_index("c")
    base = wid * b_per_w

    pltpu.sync_copy(uid_hbm.at[pl.ds(base, b_per_w)], uid_v)
    pltpu.sync_copy(mid_hbm.at[pl.ds(base, b_per_w)], mid_v)

    for p in range(b_per_w // _HALF):
        p0 = p * _HALF

        def fire(g, _):
            k0 = g * 16
            rv = uid_v[pl.ds(p0 + k0, 16)]
            rm = mid_v[pl.ds(p0 + k0, 16)]
            for k in range(16):
                pltpu.async_copy(utab_hbm.at[pl.ds(rv[k], 1), :],
                                 du_v.at[pl.ds(k0 + k, 1), :], sem)
                pltpu.async_copy(mtab_hbm.at[pl.ds(rm[k], 1), :],
                                 dm_v.at[pl.ds(k0 + k, 1), :], sem)
            return 0

        lax.fori_loop(0, _HALF // 16, fire, 0)

        pltpu.make_async_copy(utab_hbm.at[pl.ds(0, _HALF), :], du_v,
                              sem).wait()
        pltpu.make_async_copy(mtab_hbm.at[pl.ds(0, _HALF), :], dm_v,
                              sem).wait()

        pltpu.sync_copy(du_v, gu_hbm.at[pl.ds(base + p0, _HALF)])
        pltpu.sync_copy(dm_v, gm_hbm.at[pl.ds(base + p0, _HALF)])


def _dot_block(gu_ref, gm_ref, out_ref):
    out_ref[...] = jnp.sum(gu_ref[...] * gm_ref[...], axis=1)


def kernel(user_ids, movie_ids, user_table, movie_table):
    batch = user_ids.shape[0]
    info = plsc.get_sparse_core_info()
    nw = info.num_cores * info.num_subcores
    b_per_w = batch // nw
    mesh = plsc.VectorSubcoreMesh(core_axis_name="c", subcore_axis_name="s")

    gather = pl.kernel(
        functools.partial(_gather_kernel, b_per_w=b_per_w,
                          num_cores=info.num_cores),
        mesh=mesh,
        out_type=(
            jax.ShapeDtypeStruct((batch, _EMBED), jnp.float32),
            jax.ShapeDtypeStruct((batch, _EMBED), jnp.float32),
        ),
        scratch_types=[
            pltpu.VMEM((b_per_w,), jnp.int32),
            pltpu.VMEM((b_per_w,), jnp.int32),
            pltpu.VMEM((_HALF, _EMBED), jnp.float32),
            pltpu.VMEM((_HALF, _EMBED), jnp.float32),
            pltpu.SemaphoreType.DMA,
        ],
    )
    gu, gm = gather(user_ids.astype(jnp.int32), movie_ids.astype(jnp.int32),
                    user_table, movie_table)

    blk = 2048
    dot = pl.pallas_call(
        _dot_block,
        grid=(batch // blk,),
        in_specs=[
            pl.BlockSpec((blk, _EMBED), lambda i: (i, 0)),
            pl.BlockSpec((blk, _EMBED), lambda i: (i, 0)),
        ],
        out_specs=pl.BlockSpec((blk,), lambda i: (i,)),
        out_shape=jax.ShapeDtypeStruct((batch,), jnp.float32),
    )
    return dot(gu, gm)


# speedup vs baseline: 1.0462x; 1.0462x over previous
import functools

import jax
import jax.numpy as jnp
from jax import lax
from jax.experimental import pallas as pl
from jax.experimental.pallas import tpu as pltpu
from jax.experimental.pallas import tpu_sc as plsc

_EMBED = 32
_HALF = 256


def _dot_kernel(uid_hbm, mid_hbm, utab_hbm, mtab_hbm, out_hbm,
                uid_v, mid_v, du_v, dm_v, out_v, sem,
                *, b_per_w, num_cores):
    wid = lax.axis_index("s") * num_cores + lax.axis_index("c")
    base = wid * b_per_w

    pltpu.sync_copy(uid_hbm.at[pl.ds(base, b_per_w)], uid_v)
    pltpu.sync_copy(mid_hbm.at[pl.ds(base, b_per_w)], mid_v)

    lane = lax.iota(jnp.int32, 16)

    for p in range(b_per_w // _HALF):
        p0 = p * _HALF

        def fire(g, _):
            k0 = g * 16
            rv = uid_v[pl.ds(p0 + k0, 16)]
            rm = mid_v[pl.ds(p0 + k0, 16)]
            for k in range(16):
                pltpu.async_copy(utab_hbm.at[pl.ds(rv[k], 1), :],
                                 du_v.at[pl.ds(k0 + k, 1), :], sem)
                pltpu.async_copy(mtab_hbm.at[pl.ds(rm[k], 1), :],
                                 dm_v.at[pl.ds(k0 + k, 1), :], sem)
            return 0

        lax.fori_loop(0, _HALF // 16, fire, 0)

        pltpu.make_async_copy(utab_hbm.at[pl.ds(0, _HALF), :], du_v,
                              sem).wait()
        pltpu.make_async_copy(mtab_hbm.at[pl.ds(0, _HALF), :], dm_v,
                              sem).wait()

        def group(g, _):
            k0 = g * 16
            acc = jnp.zeros((16,), jnp.float32)
            for k in range(16):
                row = k0 + k
                v = (du_v[row, pl.ds(0, 16)] * dm_v[row, pl.ds(0, 16)]
                     + du_v[row, pl.ds(16, 16)] * dm_v[row, pl.ds(16, 16)])
                acc = jnp.where(lane == k, jnp.sum(v), acc)
            out_v[pl.ds(p0 + k0, 16)] = acc
            return 0

        lax.fori_loop(0, _HALF // 16, group, 0)

    pltpu.sync_copy(out_v, out_hbm.at[pl.ds(base, b_per_w)])


def kernel(user_ids, movie_ids, user_table, movie_table):
    batch = user_ids.shape[0]
    info = plsc.get_sparse_core_info()
    nw = info.num_cores * info.num_subcores
    b_per_w = batch // nw
    mesh = plsc.VectorSubcoreMesh(core_axis_name="c", subcore_axis_name="s")

    run = pl.kernel(
        functools.partial(_dot_kernel, b_per_w=b_per_w,
                          num_cores=info.num_cores),
        mesh=mesh,
        compiler_params=pltpu.CompilerParams(needs_layout_passes=False),
        out_type=jax.ShapeDtypeStruct((batch,), jnp.float32),
        scratch_types=[
            pltpu.VMEM((b_per_w,), jnp.int32),
            pltpu.VMEM((b_per_w,), jnp.int32),
            pltpu.VMEM((_HALF, _EMBED), jnp.float32),
            pltpu.VMEM((_HALF, _EMBED), jnp.float32),
            pltpu.VMEM((b_per_w,), jnp.float32),
            pltpu.SemaphoreType.DMA,
        ],
    )
    return run(user_ids.astype(jnp.int32), movie_ids.astype(jnp.int32),
               user_table, movie_table)
